# baseline (speedup 1.0000x reference)
import functools

import jax
import jax.numpy as jnp
from jax import lax
from jax.experimental import pallas as pl
from jax.experimental.pallas import tpu as pltpu
from jax.experimental.pallas import tpu_sc as plsc

_B = 16384
_D = 64
_N = 1000000
_NC, _NS, _L = 2, 16, 16
_NW = _NC * _NS
_BPW = _B // _NW
_CH = 128
_NCH = _BPW // _CH
_BN = 2048


def _transpose_body(x_ref, o_ref):
    eye = jnp.eye(_D, dtype=jnp.float32)
    o_ref[...] = jax.lax.dot_general(
        x_ref[...], eye,
        dimension_numbers=(((0,), (0,)), ((), ())),
        precision=jax.lax.Precision.HIGHEST,
    )


def _relayout(x_t):
    n = x_t.shape[1]
    grid = (n + _BN - 1) // _BN
    return pl.pallas_call(
        _transpose_body,
        out_shape=jax.ShapeDtypeStruct((n, _D), jnp.float32),
        grid=(grid,),
        in_specs=[pl.BlockSpec((_D, _BN), lambda j: (0, j))],
        out_specs=pl.BlockSpec((_BN, _D), lambda j: (j, 0)),
        compiler_params=pltpu.CompilerParams(
            dimension_semantics=("arbitrary",),
        ),
    )(x_t)


def _sc_body(uidx_hbm, iidx_hbm, eu_hbm, ei_hbm, bias_hbm, out_hbm,
             uidx_v, iidx_v, urows_v, irows_v, bias_v, out_v, sem):
    wid = lax.axis_index("s") * _NC + lax.axis_index("c")
    base = wid * _BPW
    pltpu.sync_copy(uidx_hbm.at[pl.ds(base, _BPW)], uidx_v)
    pltpu.sync_copy(iidx_hbm.at[pl.ds(base, _BPW)], iidx_v)

    copies = []
    for j in range(_NCH):
        s = pl.ds(j * _CH, _CH)
        copies.append(pltpu.async_copy(eu_hbm.at[uidx_v.at[s]], urows_v.at[s], sem))
        copies.append(pltpu.async_copy(ei_hbm.at[iidx_v.at[s]], irows_v.at[s], sem))
        copies.append(pltpu.async_copy(bias_hbm.at[iidx_v.at[s]], bias_v.at[s], sem))
    for c in copies:
        c.wait()

    iota16 = lax.iota(jnp.int32, _L)

    def group_body(g, carry):
        rows = g * _L + iota16
        bias16 = bias_v[pl.ds(g * _L, _L)]

        def d_body(d, acc):
            cols = jnp.full((_L,), d, jnp.int32)
            u = plsc.load_gather(urows_v, [rows, cols])
            it = plsc.load_gather(irows_v, [rows, cols])
            return acc + u * it

        acc = lax.fori_loop(0, _D, d_body, bias16)
        out_v[pl.ds(g * _L, _L)] = acc
        return carry

    lax.fori_loop(0, _BPW // _L, group_body, 0)
    pltpu.sync_copy(out_v, out_hbm.at[pl.ds(base, _BPW)])


def kernel(user_indices, item_indices, embedding_user, embedding_item, bias_item):
    ui = user_indices.astype(jnp.int32)
    ii = item_indices.astype(jnp.int32)
    eu = _relayout(embedding_user.T)
    ei = _relayout(embedding_item.T)
    mesh = plsc.VectorSubcoreMesh(core_axis_name="c", subcore_axis_name="s")
    f = pl.kernel(
        _sc_body,
        out_type=jax.ShapeDtypeStruct((_B,), jnp.float32),
        mesh=mesh,
        compiler_params=pltpu.CompilerParams(
            needs_layout_passes=False, use_tc_tiling_on_sc=False
        ),
        scratch_types=[
            pltpu.VMEM((_BPW,), jnp.int32),
            pltpu.VMEM((_BPW,), jnp.int32),
            pltpu.VMEM((_BPW, _D), jnp.float32),
            pltpu.VMEM((_BPW, _D), jnp.float32),
            pltpu.VMEM((_BPW,), jnp.float32),
            pltpu.VMEM((_BPW,), jnp.float32),
            pltpu.SemaphoreType.DMA,
        ],
    )
    return f(ui, ii, eu, ei, bias_item.reshape(-1))


# speedup vs baseline: 1.1412x; 1.1412x over previous
import functools

import jax
import jax.numpy as jnp
from jax import lax
from jax.experimental import pallas as pl
from jax.experimental.pallas import tpu as pltpu
from jax.experimental.pallas import tpu_sc as plsc

_B = 16384
_D = 64
_N = 1000000
_NC, _NS, _L = 2, 16, 16
_NW = _NC * _NS
_BPW = _B // _NW
_CH = 128
_NCH = _BPW // _CH
_BN = 2048


def _transpose_body(x_ref, o_ref):
    eye = jnp.eye(_D, dtype=jnp.float32)
    o_ref[...] = jax.lax.dot_general(
        x_ref[...], eye,
        dimension_numbers=(((0,), (0,)), ((), ())),
    )


def _relayout(x_t):
    n = x_t.shape[1]
    grid = (n + _BN - 1) // _BN
    return pl.pallas_call(
        _transpose_body,
        out_shape=jax.ShapeDtypeStruct((n, _D), jnp.float32),
        grid=(grid,),
        in_specs=[pl.BlockSpec((_D, _BN), lambda j: (0, j))],
        out_specs=pl.BlockSpec((_BN, _D), lambda j: (j, 0)),
        compiler_params=pltpu.CompilerParams(
            dimension_semantics=("arbitrary",),
        ),
    )(x_t)


def _sc_body(uidx_hbm, iidx_hbm, eu_hbm, ei_hbm, bias_hbm, out_hbm,
             uidx_v, iidx_v, urows_v, irows_v, bias_v, out_v, sem):
    wid = lax.axis_index("s") * _NC + lax.axis_index("c")
    base = wid * _BPW
    pltpu.sync_copy(uidx_hbm.at[pl.ds(base, _BPW)], uidx_v)
    pltpu.sync_copy(iidx_hbm.at[pl.ds(base, _BPW)], iidx_v)

    copies = []
    for j in range(_NCH):
        s = pl.ds(j * _CH, _CH)
        copies.append(pltpu.async_copy(eu_hbm.at[uidx_v.at[s]], urows_v.at[s], sem))
        copies.append(pltpu.async_copy(ei_hbm.at[iidx_v.at[s]], irows_v.at[s], sem))
        copies.append(pltpu.async_copy(bias_hbm.at[iidx_v.at[s]], bias_v.at[s], sem))
    for c in copies:
        c.wait()

    iota16 = lax.iota(jnp.int32, _L)

    def group_body(g, carry):
        rows = g * _L + iota16
        bias16 = bias_v[pl.ds(g * _L, _L)]

        def d_body(d, acc):
            cols = jnp.full((_L,), d, jnp.int32)
            u = plsc.load_gather(urows_v, [rows, cols])
            it = plsc.load_gather(irows_v, [rows, cols])
            return acc + u * it

        acc = lax.fori_loop(0, _D, d_body, bias16)
        out_v[pl.ds(g * _L, _L)] = acc
        return carry

    lax.fori_loop(0, _BPW // _L, group_body, 0)
    pltpu.sync_copy(out_v, out_hbm.at[pl.ds(base, _BPW)])


def kernel(user_indices, item_indices, embedding_user, embedding_item, bias_item):
    ui = user_indices.astype(jnp.int32)
    ii = item_indices.astype(jnp.int32)
    eu = _relayout(embedding_user.T)
    ei = _relayout(embedding_item.T)
    mesh = plsc.VectorSubcoreMesh(core_axis_name="c", subcore_axis_name="s")
    f = pl.kernel(
        _sc_body,
        out_type=jax.ShapeDtypeStruct((_B,), jnp.float32),
        mesh=mesh,
        compiler_params=pltpu.CompilerParams(
            needs_layout_passes=False, use_tc_tiling_on_sc=False
        ),
        scratch_types=[
            pltpu.VMEM((_BPW,), jnp.int32),
            pltpu.VMEM((_BPW,), jnp.int32),
            pltpu.VMEM((_BPW, _D), jnp.float32),
            pltpu.VMEM((_BPW, _D), jnp.float32),
            pltpu.VMEM((_BPW,), jnp.float32),
            pltpu.VMEM((_BPW,), jnp.float32),
            pltpu.SemaphoreType.DMA,
        ],
    )
    return f(ui, ii, eu, ei, bias_item.reshape(-1))


# speedup vs baseline: 1.3607x; 1.1924x over previous
import functools

import jax
import jax.numpy as jnp
from jax import lax
from jax.experimental import pallas as pl
from jax.experimental.pallas import tpu as pltpu
from jax.experimental.pallas import tpu_sc as plsc

_B = 16384
_D = 64
_N = 1000000
_NC, _NS, _L = 2, 16, 16
_NW = _NC * _NS
_BPW = _B // _NW
_CH = 128
_NCH = _BPW // _CH
_BN = 4096


def _transpose_body(x_ref, o_ref):
    o_ref[...] = jnp.swapaxes(x_ref[...], 0, 1)


def _relayout(x_t):
    n = x_t.shape[1]
    grid = (n + _BN - 1) // _BN
    return pl.pallas_call(
        _transpose_body,
        out_shape=jax.ShapeDtypeStruct((n, _D), jnp.float32),
        grid=(grid,),
        in_specs=[pl.BlockSpec((_D, _BN), lambda j: (0, j))],
        out_specs=pl.BlockSpec((_BN, _D), lambda j: (j, 0)),
        compiler_params=pltpu.CompilerParams(
            dimension_semantics=("arbitrary",),
        ),
    )(x_t)


def _sc_body(uidx_hbm, iidx_hbm, eu_hbm, ei_hbm, bias_hbm, out_hbm,
             uidx_v, iidx_v, urows_v, irows_v, bias_v, out_v, sem):
    wid = lax.axis_index("s") * _NC + lax.axis_index("c")
    base = wid * _BPW
    pltpu.sync_copy(uidx_hbm.at[pl.ds(base, _BPW)], uidx_v)
    pltpu.sync_copy(iidx_hbm.at[pl.ds(base, _BPW)], iidx_v)

    copies = []
    for j in range(_NCH):
        s = pl.ds(j * _CH, _CH)
        copies.append(pltpu.async_copy(eu_hbm.at[uidx_v.at[s]], urows_v.at[s], sem))
        copies.append(pltpu.async_copy(ei_hbm.at[iidx_v.at[s]], irows_v.at[s], sem))
        copies.append(pltpu.async_copy(bias_hbm.at[iidx_v.at[s]], bias_v.at[s], sem))
    for c in copies:
        c.wait()

    iota16 = lax.iota(jnp.int32, _L)

    def group_body(g, carry):
        rows = g * _L + iota16
        bias16 = bias_v[pl.ds(g * _L, _L)]

        def d_body(d, acc):
            cols = jnp.full((_L,), d, jnp.int32)
            u = plsc.load_gather(urows_v, [rows, cols])
            it = plsc.load_gather(irows_v, [rows, cols])
            return acc + u * it

        acc = lax.fori_loop(0, _D, d_body, bias16)
        out_v[pl.ds(g * _L, _L)] = acc
        return carry

    lax.fori_loop(0, _BPW // _L, group_body, 0)
    pltpu.sync_copy(out_v, out_hbm.at[pl.ds(base, _BPW)])


def kernel(user_indices, item_indices, embedding_user, embedding_item, bias_item):
    ui = user_indices.astype(jnp.int32)
    ii = item_indices.astype(jnp.int32)
    eu = _relayout(embedding_user.T)
    ei = _relayout(embedding_item.T)
    mesh = plsc.VectorSubcoreMesh(core_axis_name="c", subcore_axis_name="s")
    f = pl.kernel(
        _sc_body,
        out_type=jax.ShapeDtypeStruct((_B,), jnp.float32),
        mesh=mesh,
        compiler_params=pltpu.CompilerParams(
            needs_layout_passes=False, use_tc_tiling_on_sc=False
        ),
        scratch_types=[
            pltpu.VMEM((_BPW,), jnp.int32),
            pltpu.VMEM((_BPW,), jnp.int32),
            pltpu.VMEM((_BPW, _D), jnp.float32),
            pltpu.VMEM((_BPW, _D), jnp.float32),
            pltpu.VMEM((_BPW,), jnp.float32),
            pltpu.VMEM((_BPW,), jnp.float32),
            pltpu.SemaphoreType.DMA,
        ],
    )
    return f(ui, ii, eu, ei, bias_item.reshape(-1))


# speedup vs baseline: 1.5468x; 1.1368x over previous
import functools

import jax
import jax.numpy as jnp
from jax import lax
from jax.experimental import pallas as pl
from jax.experimental.pallas import tpu as pltpu
from jax.experimental.pallas import tpu_sc as plsc

_B = 16384
_D = 64
_N = 1000000
_NC, _NS, _L = 2, 16, 16
_NW = _NC * _NS
_BPW = _B // _NW
_CH = 128
_NCH = _BPW // _CH
_BN = 16384


def _transpose_body(x_ref, o_ref):
    o_ref[...] = jnp.swapaxes(x_ref[...], 0, 1)


def _relayout(x_t):
    n = x_t.shape[1]
    grid = (n + _BN - 1) // _BN
    return pl.pallas_call(
        _transpose_body,
        out_shape=jax.ShapeDtypeStruct((n, _D), jnp.float32),
        grid=(grid,),
        in_specs=[pl.BlockSpec((_D, _BN), lambda j: (0, j))],
        out_specs=pl.BlockSpec((_BN, _D), lambda j: (j, 0)),
        compiler_params=pltpu.CompilerParams(
            dimension_semantics=("arbitrary",),
        ),
    )(x_t)


def _sc_body(uidx_hbm, iidx_hbm, eu_hbm, ei_hbm, bias_hbm, out_hbm,
             uidx_v, iidx_v, urows_v, irows_v, bias_v, out_v, sem):
    wid = lax.axis_index("s") * _NC + lax.axis_index("c")
    base = wid * _BPW
    pltpu.sync_copy(uidx_hbm.at[pl.ds(base, _BPW)], uidx_v)
    pltpu.sync_copy(iidx_hbm.at[pl.ds(base, _BPW)], iidx_v)

    copies = []
    for j in range(_NCH):
        s = pl.ds(j * _CH, _CH)
        copies.append(pltpu.async_copy(eu_hbm.at[uidx_v.at[s]], urows_v.at[s], sem))
        copies.append(pltpu.async_copy(ei_hbm.at[iidx_v.at[s]], irows_v.at[s], sem))
        copies.append(pltpu.async_copy(bias_hbm.at[iidx_v.at[s]], bias_v.at[s], sem))
    for c in copies:
        c.wait()

    iota16 = lax.iota(jnp.int32, _L)

    def group_body(g, carry):
        rows = g * _L + iota16
        bias16 = bias_v[pl.ds(g * _L, _L)]

        def d_body(d, acc):
            cols = jnp.full((_L,), d, jnp.int32)
            u = plsc.load_gather(urows_v, [rows, cols])
            it = plsc.load_gather(irows_v, [rows, cols])
            return acc + u * it

        acc = lax.fori_loop(0, _D, d_body, bias16)
        out_v[pl.ds(g * _L, _L)] = acc
        return carry

    lax.fori_loop(0, _BPW // _L, group_body, 0)
    pltpu.sync_copy(out_v, out_hbm.at[pl.ds(base, _BPW)])


def kernel(user_indices, item_indices, embedding_user, embedding_item, bias_item):
    ui = user_indices.astype(jnp.int32)
    ii = item_indices.astype(jnp.int32)
    eu = _relayout(embedding_user.T)
    ei = _relayout(embedding_item.T)
    mesh = plsc.VectorSubcoreMesh(core_axis_name="c", subcore_axis_name="s")
    f = pl.kernel(
        _sc_body,
        out_type=jax.ShapeDtypeStruct((_B,), jnp.float32),
        mesh=mesh,
        compiler_params=pltpu.CompilerParams(
            needs_layout_passes=False, use_tc_tiling_on_sc=False
        ),
        scratch_types=[
            pltpu.VMEM((_BPW,), jnp.int32),
            pltpu.VMEM((_BPW,), jnp.int32),
            pltpu.VMEM((_BPW, _D), jnp.float32),
            pltpu.VMEM((_BPW, _D), jnp.float32),
            pltpu.VMEM((_BPW,), jnp.float32),
            pltpu.VMEM((_BPW,), jnp.float32),
            pltpu.SemaphoreType.DMA,
        ],
    )
    return f(ui, ii, eu, ei, bias_item.reshape(-1))


# speedup vs baseline: 1.6263x; 1.0514x over previous
import functools

import jax
import jax.numpy as jnp
from jax import lax
from jax.experimental import pallas as pl
from jax.experimental.pallas import tpu as pltpu
from jax.experimental.pallas import tpu_sc as plsc

_B = 16384
_D = 64
_N = 1000000
_NC, _NS, _L = 2, 16, 16
_NW = _NC * _NS
_BPW = _B // _NW
_CH = 128
_NCH = _BPW // _CH
_BN = 16384


def _transpose_body(x_ref, o_ref):
    o_ref[...] = jnp.swapaxes(x_ref[...], 0, 1)


def _relayout(x_t):
    n = x_t.shape[1]
    grid = (n + _BN - 1) // _BN
    return pl.pallas_call(
        _transpose_body,
        out_shape=jax.ShapeDtypeStruct((n, _D), jnp.float32),
        grid=(grid,),
        in_specs=[pl.BlockSpec((_D, _BN), lambda j: (0, j))],
        out_specs=pl.BlockSpec((_BN, _D), lambda j: (j, 0)),
        compiler_params=pltpu.CompilerParams(
            dimension_semantics=("arbitrary",),
        ),
    )(x_t)


def _sc_body(uidx_hbm, iidx_hbm, eu_hbm, ei_hbm, bias_hbm, out_hbm,
             uidx_v, iidx_v, urows_v, irows_v, bias_v, out_v, sem):
    wid = lax.axis_index("s") * _NC + lax.axis_index("c")
    base = wid * _BPW
    pltpu.sync_copy(uidx_hbm.at[pl.ds(base, _BPW)], uidx_v)
    pltpu.sync_copy(iidx_hbm.at[pl.ds(base, _BPW)], iidx_v)

    copies = []
    for j in range(_NCH):
        s = pl.ds(j * _CH, _CH)
        copies.append(pltpu.async_copy(eu_hbm.at[uidx_v.at[s]], urows_v.at[s], sem))
        copies.append(pltpu.async_copy(ei_hbm.at[iidx_v.at[s]], irows_v.at[s], sem))
        copies.append(pltpu.async_copy(bias_hbm.at[iidx_v.at[s]], bias_v.at[s], sem))
    for c in copies:
        c.wait()

    iota16 = lax.iota(jnp.int32, _L)

    def group_body(g, carry):
        rows = g * _L + iota16
        bias16 = bias_v[pl.ds(g * _L, _L)]

        def d_body(d, acc):
            cols = jnp.full((_L,), d, jnp.int32)
            u = plsc.load_gather(urows_v, [rows, cols])
            it = plsc.load_gather(irows_v, [rows, cols])
            return acc + u * it

        acc = lax.fori_loop(0, _D, d_body, bias16)
        out_v[pl.ds(g * _L, _L)] = acc
        return carry

    lax.fori_loop(0, _BPW // _L, group_body, 0)
    pltpu.sync_copy(out_v, out_hbm.at[pl.ds(base, _BPW)])


def kernel(user_indices, item_indices, embedding_user, embedding_item, bias_item):
    ui = user_indices.astype(jnp.int32)
    ii = item_indices.astype(jnp.int32)
    eu = _relayout(embedding_user.T)
    ei = embedding_item
    mesh = plsc.VectorSubcoreMesh(core_axis_name="c", subcore_axis_name="s")
    f = pl.kernel(
        _sc_body,
        out_type=jax.ShapeDtypeStruct((_B,), jnp.float32),
        mesh=mesh,
        compiler_params=pltpu.CompilerParams(
            needs_layout_passes=False, use_tc_tiling_on_sc=False
        ),
        scratch_types=[
            pltpu.VMEM((_BPW,), jnp.int32),
            pltpu.VMEM((_BPW,), jnp.int32),
            pltpu.VMEM((_BPW, _D), jnp.float32),
            pltpu.VMEM((_BPW, _D), jnp.float32),
            pltpu.VMEM((_BPW,), jnp.float32),
            pltpu.VMEM((_BPW,), jnp.float32),
            pltpu.SemaphoreType.DMA,
        ],
    )
    return f(ui, ii, eu, ei, bias_item.reshape(-1))


# speedup vs baseline: 4.0608x; 2.4969x over previous
import jax
import jax.numpy as jnp
from jax import lax
from jax.experimental import pallas as pl
from jax.experimental.pallas import tpu as pltpu
from jax.experimental.pallas import tpu_sc as plsc

_B = 16384
_D = 64
_N = 1000000
_NC, _NS, _L = 2, 16, 16
_NW = _NC * _NS
_BPW = _B // _NW
_CH = 128
_NCH = _BPW // _CH


def _bias_body(iidx_hbm, bias_hbm, out_hbm, iidx_v, bias_v, sem):
    wid = lax.axis_index("s") * _NC + lax.axis_index("c")
    base = wid * _BPW
    pltpu.sync_copy(iidx_hbm.at[pl.ds(base, _BPW)], iidx_v)
    copies = []
    for j in range(_NCH):
        s = pl.ds(j * _CH, _CH)
        copies.append(pltpu.async_copy(bias_hbm.at[iidx_v.at[s]], bias_v.at[s], sem))
    for c in copies:
        c.wait()
    pltpu.sync_copy(bias_v, out_hbm.at[pl.ds(base, _BPW)])


def _main_body(uidx_hbm, iidx_hbm, eu_hbm, ei_hbm, b16_hbm, out_hbm,
               uidx_s, iidx_s, u_win, i_win, bias_v, out_v, usem, isem):
    wid = lax.axis_index("s") * _NC + lax.axis_index("c")
    base = wid * _BPW
    pltpu.sync_copy(uidx_hbm.at[pl.ds(base, _BPW)], uidx_s)
    pltpu.sync_copy(iidx_hbm.at[pl.ds(base, _BPW)], iidx_s)
    pltpu.sync_copy(b16_hbm.at[pl.ds(base, _BPW)], bias_v)

    iota16 = lax.iota(jnp.int32, _L)

    def issue(nu, ni, slot):
        cu = lax.shift_right_logical(nu, 7) * 128
        ci = lax.shift_right_logical(ni, 7) * 128
        pltpu.async_copy(eu_hbm.at[:, pl.ds(cu, 128)], u_win.at[slot], usem)
        pltpu.async_copy(ei_hbm.at[:, pl.ds(ci, 128)], i_win.at[slot], isem)

    uv0 = uidx_s[pl.ds(0, _L)]
    iv0 = iidx_s[pl.ds(0, _L)]
    issue(uv0[0], iv0[0], 0)

    def group_body(g, carry):
        uvec = uidx_s[pl.ds(g * _L, _L)]
        ivec = iidx_s[pl.ds(g * _L, _L)]
        gn = jnp.minimum(g + 1, _BPW // _L - 1)
        uvec_n = uidx_s[pl.ds(gn * _L, _L)]
        ivec_n = iidx_s[pl.ds(gn * _L, _L)]
        res = bias_v[pl.ds(g * _L, _L)]
        for k in range(_L):
            slot = k & 1
            if k < _L - 1:
                issue(uvec[k + 1], ivec[k + 1], 1 - slot)
            else:
                @pl.when(g < _BPW // _L - 1)
                def _():
                    issue(uvec_n[0], ivec_n[0], 1 - slot)

            pltpu.make_async_copy(eu_hbm.at[:, pl.ds(0, 128)], u_win.at[slot], usem).wait()
            pltpu.make_async_copy(ei_hbm.at[:, pl.ds(0, 128)], i_win.at[slot], isem).wait()

            lu = jnp.full((_L,), uvec[k] & 127, jnp.int32)
            li = jnp.full((_L,), ivec[k] & 127, jnp.int32)
            acc = jnp.zeros((_L,), jnp.float32)
            for kk in range(_D // _L):
                rows = kk * _L + iota16
                u = plsc.load_gather(u_win.at[slot], [rows, lu])
                v = plsc.load_gather(i_win.at[slot], [rows, li])
                acc = acc + u * v
            s = jnp.sum(acc)
            res = jnp.where(iota16 == k, res + s, res)
        out_v[pl.ds(g * _L, _L)] = res
        return carry

    lax.fori_loop(0, _BPW // _L, group_body, 0)
    pltpu.sync_copy(out_v, out_hbm.at[pl.ds(base, _BPW)])


def kernel(user_indices, item_indices, embedding_user, embedding_item, bias_item):
    ui = user_indices.astype(jnp.int32)
    ii = item_indices.astype(jnp.int32)
    mesh = plsc.VectorSubcoreMesh(core_axis_name="c", subcore_axis_name="s")

    bias16 = pl.kernel(
        _bias_body,
        out_type=jax.ShapeDtypeStruct((_B,), jnp.float32),
        mesh=mesh,
        compiler_params=pltpu.CompilerParams(
            needs_layout_passes=False, use_tc_tiling_on_sc=False
        ),
        scratch_types=[
            pltpu.VMEM((_BPW,), jnp.int32),
            pltpu.VMEM((_BPW,), jnp.float32),
            pltpu.SemaphoreType.DMA,
        ],
    )(ii, bias_item.reshape(-1))

    out = pl.kernel(
        _main_body,
        out_type=jax.ShapeDtypeStruct((_B,), jnp.float32),
        mesh=mesh,
        compiler_params=pltpu.CompilerParams(needs_layout_passes=False),
        scratch_types=[
            pltpu.VMEM((_BPW,), jnp.int32),
            pltpu.VMEM((_BPW,), jnp.int32),
            pltpu.VMEM((2, _D, 128), jnp.float32),
            pltpu.VMEM((2, _D, 128), jnp.float32),
            pltpu.VMEM((_BPW,), jnp.float32),
            pltpu.VMEM((_BPW,), jnp.float32),
            pltpu.SemaphoreType.DMA,
            pltpu.SemaphoreType.DMA,
        ],
    )(ui, ii, embedding_user.T, embedding_item.T, bias16)
    return out


# speedup vs baseline: 4.7835x; 1.1780x over previous
import jax
import jax.numpy as jnp
from jax import lax
from jax.experimental import pallas as pl
from jax.experimental.pallas import tpu as pltpu
from jax.experimental.pallas import tpu_sc as plsc

_B = 16384
_D = 64
_N = 1000000
_NC, _NS, _L = 2, 16, 16
_NW = _NC * _NS
_BPW = _B // _NW
_CH = 128
_NCH = _BPW // _CH


def _bias_body(iidx_hbm, bias_hbm, out_hbm, iidx_v, bias_v, sem):
    wid = lax.axis_index("s") * _NC + lax.axis_index("c")
    base = wid * _BPW
    pltpu.sync_copy(iidx_hbm.at[pl.ds(base, _BPW)], iidx_v)
    copies = []
    for j in range(_NCH):
        s = pl.ds(j * _CH, _CH)
        copies.append(pltpu.async_copy(bias_hbm.at[iidx_v.at[s]], bias_v.at[s], sem))
    for c in copies:
        c.wait()
    pltpu.sync_copy(bias_v, out_hbm.at[pl.ds(base, _BPW)])


def _main_body(uidx_hbm, iidx_hbm, eu_hbm, ei_hbm, b16_hbm, out_hbm,
               uidx_s, iidx_s, u_win, i_win, bias_v, out_v, usem, isem):
    wid = lax.axis_index("s") * _NC + lax.axis_index("c")
    base = wid * _BPW
    pltpu.sync_copy(uidx_hbm.at[pl.ds(base, _BPW)], uidx_s)
    pltpu.sync_copy(iidx_hbm.at[pl.ds(base, _BPW)], iidx_s)
    pltpu.sync_copy(b16_hbm.at[pl.ds(base, _BPW)], bias_v)

    iota16 = lax.iota(jnp.int32, _L)

    def issue(nu, ni, slot):
        cu = lax.shift_right_logical(nu, 7) * 128
        ci = lax.shift_right_logical(ni, 7) * 128
        pltpu.async_copy(eu_hbm.at[:, pl.ds(cu, 128)], u_win.at[slot], usem)
        pltpu.async_copy(ei_hbm.at[:, pl.ds(ci, 128)], i_win.at[slot], isem)

    _NBUF = 4
    _AHEAD = _NBUF - 1
    uv0 = uidx_s[pl.ds(0, _L)]
    iv0 = iidx_s[pl.ds(0, _L)]
    for t in range(_AHEAD):
        issue(uv0[t], iv0[t], t)

    def group_body(g, carry):
        uvec = uidx_s[pl.ds(g * _L, _L)]
        ivec = iidx_s[pl.ds(g * _L, _L)]
        gn = jnp.minimum(g + 1, _BPW // _L - 1)
        uvec_n = uidx_s[pl.ds(gn * _L, _L)]
        ivec_n = iidx_s[pl.ds(gn * _L, _L)]
        res = bias_v[pl.ds(g * _L, _L)]
        for k in range(_L):
            slot = k & (_NBUF - 1)
            nslot = (k + _AHEAD) & (_NBUF - 1)
            if k + _AHEAD < _L:
                issue(uvec[k + _AHEAD], ivec[k + _AHEAD], nslot)
            else:
                @pl.when(g < _BPW // _L - 1)
                def _():
                    issue(uvec_n[k + _AHEAD - _L], ivec_n[k + _AHEAD - _L], nslot)

            pltpu.make_async_copy(eu_hbm.at[:, pl.ds(0, 128)], u_win.at[slot], usem).wait()
            pltpu.make_async_copy(ei_hbm.at[:, pl.ds(0, 128)], i_win.at[slot], isem).wait()

            lu = jnp.full((_L,), uvec[k] & 127, jnp.int32)
            li = jnp.full((_L,), ivec[k] & 127, jnp.int32)
            acc = jnp.zeros((_L,), jnp.float32)
            for kk in range(_D // _L):
                rows = kk * _L + iota16
                u = plsc.load_gather(u_win.at[slot], [rows, lu])
                v = plsc.load_gather(i_win.at[slot], [rows, li])
                acc = acc + u * v
            s = jnp.sum(acc)
            res = jnp.where(iota16 == k, res + s, res)
        out_v[pl.ds(g * _L, _L)] = res
        return carry

    lax.fori_loop(0, _BPW // _L, group_body, 0)
    pltpu.sync_copy(out_v, out_hbm.at[pl.ds(base, _BPW)])


def kernel(user_indices, item_indices, embedding_user, embedding_item, bias_item):
    ui = user_indices.astype(jnp.int32)
    ii = item_indices.astype(jnp.int32)
    mesh = plsc.VectorSubcoreMesh(core_axis_name="c", subcore_axis_name="s")

    bias16 = pl.kernel(
        _bias_body,
        out_type=jax.ShapeDtypeStruct((_B,), jnp.float32),
        mesh=mesh,
        compiler_params=pltpu.CompilerParams(
            needs_layout_passes=False, use_tc_tiling_on_sc=False
        ),
        scratch_types=[
            pltpu.VMEM((_BPW,), jnp.int32),
            pltpu.VMEM((_BPW,), jnp.float32),
            pltpu.SemaphoreType.DMA,
        ],
    )(ii, bias_item.reshape(-1))

    out = pl.kernel(
        _main_body,
        out_type=jax.ShapeDtypeStruct((_B,), jnp.float32),
        mesh=mesh,
        compiler_params=pltpu.CompilerParams(needs_layout_passes=False),
        scratch_types=[
            pltpu.VMEM((_BPW,), jnp.int32),
            pltpu.VMEM((_BPW,), jnp.int32),
            pltpu.VMEM((4, _D, 128), jnp.float32),
            pltpu.VMEM((4, _D, 128), jnp.float32),
            pltpu.VMEM((_BPW,), jnp.float32),
            pltpu.VMEM((_BPW,), jnp.float32),
            pltpu.SemaphoreType.DMA,
            pltpu.SemaphoreType.DMA,
        ],
    )(ui, ii, embedding_user.T, embedding_item.T, bias16)
    return out
